# baseline (speedup 1.0000x reference)
import jax
import jax.numpy as jnp
from jax.experimental import pallas as pl
from jax.experimental.pallas import tpu as pltpu
from jax.experimental.pallas import tpu_sc as plsc

BATCH = 16384
DIM = 32
N_TABLE = 1000000
PACK = 128 // DIM
OCT = 2 * PACK
WIDE = 128
TR_ROWS = 6144
TR_COLS = TR_ROWS * OCT
N_FULL = N_TABLE // TR_COLS
TAIL = N_TABLE - N_FULL * TR_COLS
N_PACKED = (N_FULL + 1) * TR_ROWS
GATHER_WINDOW = 128
MLP_BLOCK = 4096


def _pack_body(a_ref, b_ref, at_ref, bt_ref, oa_ref, ob_ref):
    is_tail = pl.program_id(0) == N_FULL
    eye = jnp.eye(DIM, dtype=jnp.float32)
    sels = [
        jnp.pad(eye, ((s * DIM, WIDE - (s + 1) * DIM), (0, 0)))
        for s in range(PACK)
    ]

    def pack_from(src_val, dst):
        planes = []
        for half in range(2):
            w = jnp.zeros((WIDE, TR_ROWS), jnp.float32)
            for s in range(PACK):
                a = 2 * s + half
                w = w + jax.lax.dot_general(
                    sels[s], src_val[:, a * TR_ROWS:(a + 1) * TR_ROWS],
                    (((1,), (0,)), ((), ())),
                    preferred_element_type=jnp.float32)
            planes.append(jax.lax.bitcast_convert_type(w.T, jnp.int32))
        lo, hi = planes
        dst[...] = jnp.bitwise_or(
            jnp.right_shift(lo, 16) & 0xFFFF, hi & ~0xFFFF)

    @pl.when(jnp.logical_not(is_tail))
    def _():
        pack_from(a_ref[...], oa_ref)
        pack_from(b_ref[...], ob_ref)

    @pl.when(is_tail)
    def _():
        pack_from(at_ref[...], oa_ref)
        pack_from(bt_ref[...], ob_ref)


def _tc_pack(ue_t, ie_t):
    ue_tail = jnp.pad(ue_t[:, N_FULL * TR_COLS:], ((0, 0), (0, TR_COLS - TAIL)))
    ie_tail = jnp.pad(ie_t[:, N_FULL * TR_COLS:], ((0, 0), (0, TR_COLS - TAIL)))
    grid = (N_FULL + 1,)
    out_sds = jax.ShapeDtypeStruct((N_PACKED, WIDE), jnp.int32)
    main_spec = pl.BlockSpec((DIM, TR_COLS), lambda i: (0, jnp.minimum(i, N_FULL - 1)))
    tail_spec = pl.BlockSpec((DIM, TR_COLS), lambda i: (0, 0))
    return pl.pallas_call(
        _pack_body,
        grid=grid,
        in_specs=[main_spec, main_spec, tail_spec, tail_spec],
        out_specs=[
            pl.BlockSpec((TR_ROWS, WIDE), lambda i: (i, 0)),
            pl.BlockSpec((TR_ROWS, WIDE), lambda i: (i, 0)),
        ],
        out_shape=(out_sds, out_sds),
        compiler_params=pltpu.CompilerParams(
            dimension_semantics=("parallel",),
            fuse_transposed_lhs_in_matmul=True),
    )(ue_t, ie_t, ue_tail, ie_tail)


def _sc_gather(ue_wide, ie_wide, uid4, iid4):
    mesh = plsc.VectorSubcoreMesh(core_axis_name="core", subcore_axis_name="subcore")
    uid = uid4.reshape(1, BATCH)
    iid = iid4.reshape(1, BATCH)
    out_t = (
        jax.ShapeDtypeStruct((BATCH, WIDE), jnp.int32),
        jax.ShapeDtypeStruct((BATCH, WIDE), jnp.int32),
    )

    @pl.kernel(out_type=out_t, mesh=mesh)
    def gather_kernel(ue_hbm, ie_hbm, uid_hbm, iid_hbm, ue_out, ie_out):
        def body(uid_vmem, iid_vmem, ue_vmem, ie_vmem):
            pltpu.sync_copy(ue_hbm.at[uid_vmem.at[0]], ue_vmem)
            pltpu.sync_copy(ie_hbm.at[iid_vmem.at[0]], ie_vmem)

        pltpu.emit_pipeline(
            body,
            grid=(BATCH // GATHER_WINDOW,),
            in_specs=[
                pl.BlockSpec((1, GATHER_WINDOW), lambda i: (0, i)),
                pl.BlockSpec((1, GATHER_WINDOW), lambda i: (0, i)),
            ],
            out_specs=[
                pl.BlockSpec((GATHER_WINDOW, WIDE), lambda i: (i, 0)),
                pl.BlockSpec((GATHER_WINDOW, WIDE), lambda i: (i, 0)),
            ],
            core_axis_name=("core", "subcore"),
            dimension_semantics=(pltpu.PARALLEL,),
        )(uid_hbm, iid_hbm, ue_out, ie_out)

    return gather_kernel(ue_wide, ie_wide, uid, iid)


def _mlp_body(wu_ref, wi_ref, qu_ref, qi_ref, w0a_ref, w0b_ref, b0_ref,
              w1_ref, b1_ref, w2_ref, b2_ref, wo_ref, bo_ref, o_ref):
    lane_slot = jax.lax.broadcasted_iota(jnp.int32, (1, WIDE), 1) // DIM

    def decode_select(w_ref, q_ref):
        w = w_ref[...]
        lo = jax.lax.bitcast_convert_type(jnp.left_shift(w, 16), jnp.float32)
        hi = jax.lax.bitcast_convert_type(w & ~0xFFFF, jnp.float32)
        q = q_ref[...]
        m_lo = ((q & 1) == 0) & (q // 2 == lane_slot)
        m_hi = ((q & 1) == 1) & (q // 2 == lane_slot)
        return lo * m_lo.astype(jnp.float32) + hi * m_hi.astype(jnp.float32)

    mu = decode_select(wu_ref, qu_ref)
    mi = decode_select(wi_ref, qi_ref)
    x = mu @ w0a_ref[...] + mi @ w0b_ref[...] + b0_ref[...]
    x = jnp.maximum(x, 0.0)
    x = jnp.maximum(x @ w1_ref[...] + b1_ref[...], 0.0)
    x = jnp.maximum(x @ w2_ref[...] + b2_ref[...], 0.0)
    o_ref[...] = jax.nn.sigmoid(x @ wo_ref[...] + bo_ref[...])


def _tc_mlp(wu, wi, qu, qi, W0, b0, W1, b1, W2, b2, Wout, bout):
    w0a = jnp.tile(W0[:DIM], (PACK, 1))
    w0b = jnp.tile(W0[DIM:], (PACK, 1))
    full = lambda shape: pl.BlockSpec(shape, lambda i: (0, 0))
    grid = (BATCH // MLP_BLOCK,)
    return pl.pallas_call(
        _mlp_body,
        grid=grid,
        in_specs=[
            pl.BlockSpec((MLP_BLOCK, WIDE), lambda i: (i, 0)),
            pl.BlockSpec((MLP_BLOCK, WIDE), lambda i: (i, 0)),
            pl.BlockSpec((MLP_BLOCK, 1), lambda i: (i, 0)),
            pl.BlockSpec((MLP_BLOCK, 1), lambda i: (i, 0)),
            full(w0a.shape),
            full(w0b.shape),
            full((1, b0.shape[0])),
            full(W1.shape),
            full((1, b1.shape[0])),
            full(W2.shape),
            full((1, b2.shape[0])),
            full(Wout.shape),
            full((1, bout.shape[0])),
        ],
        out_specs=pl.BlockSpec((MLP_BLOCK, 1), lambda i: (i, 0)),
        out_shape=jax.ShapeDtypeStruct((BATCH, 1), jnp.float32),
    )(wu, wi, qu, qi, w0a, w0b, b0.reshape(1, -1), W1, b1.reshape(1, -1),
      W2, b2.reshape(1, -1), Wout, bout.reshape(1, -1))


def kernel(user_id, item_id, user_emb, item_emb, W0, b0, W1, b1, W2, b2, Wout, bout):
    user_id = user_id.astype(jnp.int32)
    item_id = item_id.astype(jnp.int32)
    ue_wide, ie_wide = _tc_pack(user_emb.T, item_emb.T)
    uw = TR_ROWS * (user_id // TR_COLS) + user_id % TR_ROWS
    iw = TR_ROWS * (item_id // TR_COLS) + item_id % TR_ROWS
    wu, wi = _sc_gather(ue_wide, ie_wide, uw, iw)
    qu = ((user_id // TR_ROWS) % OCT).reshape(BATCH, 1)
    qi = ((item_id // TR_ROWS) % OCT).reshape(BATCH, 1)
    return _tc_mlp(wu, wi, qu, qi, W0, b0, W1, b1, W2, b2, Wout, bout)


# speedup vs baseline: 1.0115x; 1.0115x over previous
import jax
import jax.numpy as jnp
from jax.experimental import pallas as pl
from jax.experimental.pallas import tpu as pltpu
from jax.experimental.pallas import tpu_sc as plsc

BATCH = 16384
DIM = 32
N_TABLE = 1000000
PACK = 128 // DIM
OCT = 2 * PACK
WIDE = 128
TR_ROWS = 4096
TR_COLS = TR_ROWS * OCT
N_FULL = N_TABLE // TR_COLS
TAIL = N_TABLE - N_FULL * TR_COLS
N_PACKED = (N_FULL + 1) * TR_ROWS
GATHER_WINDOW = 128
MLP_BLOCK = 4096


def _pack_body(a_ref, b_ref, at_ref, bt_ref, oa_ref, ob_ref):
    is_tail = pl.program_id(0) == N_FULL
    eye = jnp.eye(DIM, dtype=jnp.float32)
    sels = [
        jnp.pad(eye, ((s * DIM, WIDE - (s + 1) * DIM), (0, 0)))
        for s in range(PACK)
    ]

    def pack_from(src_val, dst):
        planes = []
        for half in range(2):
            w = jnp.zeros((WIDE, TR_ROWS), jnp.float32)
            for s in range(PACK):
                a = 2 * s + half
                w = w + jax.lax.dot_general(
                    sels[s], src_val[:, a * TR_ROWS:(a + 1) * TR_ROWS],
                    (((1,), (0,)), ((), ())),
                    preferred_element_type=jnp.float32)
            planes.append(jax.lax.bitcast_convert_type(w.T, jnp.int32))
        lo, hi = planes
        dst[...] = jnp.bitwise_or(
            jnp.right_shift(lo, 16) & 0xFFFF, hi & ~0xFFFF)

    @pl.when(jnp.logical_not(is_tail))
    def _():
        pack_from(a_ref[...], oa_ref)
        pack_from(b_ref[...], ob_ref)

    @pl.when(is_tail)
    def _():
        pack_from(at_ref[...], oa_ref)
        pack_from(bt_ref[...], ob_ref)


def _tc_pack(ue_t, ie_t):
    ue_tail = jnp.pad(ue_t[:, N_FULL * TR_COLS:], ((0, 0), (0, TR_COLS - TAIL)))
    ie_tail = jnp.pad(ie_t[:, N_FULL * TR_COLS:], ((0, 0), (0, TR_COLS - TAIL)))
    grid = (N_FULL + 1,)
    out_sds = jax.ShapeDtypeStruct((N_PACKED, WIDE), jnp.int32)
    main_spec = pl.BlockSpec((DIM, TR_COLS), lambda i: (0, jnp.minimum(i, N_FULL - 1)))
    tail_spec = pl.BlockSpec((DIM, TR_COLS), lambda i: (0, 0))
    return pl.pallas_call(
        _pack_body,
        grid=grid,
        in_specs=[main_spec, main_spec, tail_spec, tail_spec],
        out_specs=[
            pl.BlockSpec((TR_ROWS, WIDE), lambda i: (i, 0)),
            pl.BlockSpec((TR_ROWS, WIDE), lambda i: (i, 0)),
        ],
        out_shape=(out_sds, out_sds),
        compiler_params=pltpu.CompilerParams(
            dimension_semantics=("parallel",),
            fuse_transposed_lhs_in_matmul=True),
    )(ue_t, ie_t, ue_tail, ie_tail)


def _sc_gather(ue_wide, ie_wide, uid4, iid4):
    mesh = plsc.VectorSubcoreMesh(core_axis_name="core", subcore_axis_name="subcore")
    uid = uid4.reshape(1, BATCH)
    iid = iid4.reshape(1, BATCH)
    out_t = (
        jax.ShapeDtypeStruct((BATCH, WIDE), jnp.int32),
        jax.ShapeDtypeStruct((BATCH, WIDE), jnp.int32),
    )

    @pl.kernel(out_type=out_t, mesh=mesh)
    def gather_kernel(ue_hbm, ie_hbm, uid_hbm, iid_hbm, ue_out, ie_out):
        def body(uid_vmem, iid_vmem, ue_vmem, ie_vmem):
            pltpu.sync_copy(ue_hbm.at[uid_vmem.at[0]], ue_vmem)
            pltpu.sync_copy(ie_hbm.at[iid_vmem.at[0]], ie_vmem)

        pltpu.emit_pipeline(
            body,
            grid=(BATCH // GATHER_WINDOW,),
            in_specs=[
                pl.BlockSpec((1, GATHER_WINDOW), lambda i: (0, i)),
                pl.BlockSpec((1, GATHER_WINDOW), lambda i: (0, i)),
            ],
            out_specs=[
                pl.BlockSpec((GATHER_WINDOW, WIDE), lambda i: (i, 0)),
                pl.BlockSpec((GATHER_WINDOW, WIDE), lambda i: (i, 0)),
            ],
            core_axis_name=("core", "subcore"),
            dimension_semantics=(pltpu.PARALLEL,),
        )(uid_hbm, iid_hbm, ue_out, ie_out)

    return gather_kernel(ue_wide, ie_wide, uid, iid)


def _mlp_body(wu_ref, wi_ref, qu_ref, qi_ref, w0a_ref, w0b_ref, b0_ref,
              w1_ref, b1_ref, w2_ref, b2_ref, wo_ref, bo_ref, o_ref):
    lane_slot = jax.lax.broadcasted_iota(jnp.int32, (1, WIDE), 1) // DIM

    def decode_select(w_ref, q_ref):
        w = w_ref[...]
        lo = jax.lax.bitcast_convert_type(jnp.left_shift(w, 16), jnp.float32)
        hi = jax.lax.bitcast_convert_type(w & ~0xFFFF, jnp.float32)
        q = q_ref[...]
        m_lo = ((q & 1) == 0) & (q // 2 == lane_slot)
        m_hi = ((q & 1) == 1) & (q // 2 == lane_slot)
        return lo * m_lo.astype(jnp.float32) + hi * m_hi.astype(jnp.float32)

    mu = decode_select(wu_ref, qu_ref)
    mi = decode_select(wi_ref, qi_ref)
    x = mu @ w0a_ref[...] + mi @ w0b_ref[...] + b0_ref[...]
    x = jnp.maximum(x, 0.0)
    x = jnp.maximum(x @ w1_ref[...] + b1_ref[...], 0.0)
    x = jnp.maximum(x @ w2_ref[...] + b2_ref[...], 0.0)
    o_ref[...] = jax.nn.sigmoid(x @ wo_ref[...] + bo_ref[...])


def _tc_mlp(wu, wi, qu, qi, W0, b0, W1, b1, W2, b2, Wout, bout):
    w0a = jnp.tile(W0[:DIM], (PACK, 1))
    w0b = jnp.tile(W0[DIM:], (PACK, 1))
    full = lambda shape: pl.BlockSpec(shape, lambda i: (0, 0))
    grid = (BATCH // MLP_BLOCK,)
    return pl.pallas_call(
        _mlp_body,
        grid=grid,
        in_specs=[
            pl.BlockSpec((MLP_BLOCK, WIDE), lambda i: (i, 0)),
            pl.BlockSpec((MLP_BLOCK, WIDE), lambda i: (i, 0)),
            pl.BlockSpec((MLP_BLOCK, 1), lambda i: (i, 0)),
            pl.BlockSpec((MLP_BLOCK, 1), lambda i: (i, 0)),
            full(w0a.shape),
            full(w0b.shape),
            full((1, b0.shape[0])),
            full(W1.shape),
            full((1, b1.shape[0])),
            full(W2.shape),
            full((1, b2.shape[0])),
            full(Wout.shape),
            full((1, bout.shape[0])),
        ],
        out_specs=pl.BlockSpec((MLP_BLOCK, 1), lambda i: (i, 0)),
        out_shape=jax.ShapeDtypeStruct((BATCH, 1), jnp.float32),
    )(wu, wi, qu, qi, w0a, w0b, b0.reshape(1, -1), W1, b1.reshape(1, -1),
      W2, b2.reshape(1, -1), Wout, bout.reshape(1, -1))


def kernel(user_id, item_id, user_emb, item_emb, W0, b0, W1, b1, W2, b2, Wout, bout):
    user_id = user_id.astype(jnp.int32)
    item_id = item_id.astype(jnp.int32)
    ue_wide, ie_wide = _tc_pack(user_emb.T, item_emb.T)
    uw = TR_ROWS * (user_id // TR_COLS) + user_id % TR_ROWS
    iw = TR_ROWS * (item_id // TR_COLS) + item_id % TR_ROWS
    wu, wi = _sc_gather(ue_wide, ie_wide, uw, iw)
    qu = ((user_id // TR_ROWS) % OCT).reshape(BATCH, 1)
    qi = ((item_id // TR_ROWS) % OCT).reshape(BATCH, 1)
    return _tc_mlp(wu, wi, qu, qi, W0, b0, W1, b1, W2, b2, Wout, bout)
